# baseline (device time: 11036 ns/iter reference)
import jax
import jax.numpy as jnp
from jax import lax
from jax.experimental import pallas as pl
from jax.experimental.pallas import tpu as pltpu

N_DEV = 4


def kernel(A, B):
    m, _ = A.shape
    _, n = B.shape
    m_out = m // N_DEV

    def body(a_ref, b_ref, out_ref, part_ref, send_ref, recv_ref,
             send_sems, recv_sems):
        p = lax.axis_index("i")

        part_ref[:, :] = jnp.dot(
            a_ref[:, :].astype(jnp.bfloat16),
            b_ref[:, :].astype(jnp.bfloat16),
            preferred_element_type=jnp.float32,
        )

        def chunk_partial(c):
            return part_ref[pl.ds(c * m_out, m_out), :]

        for d in (1, 2, 3):
            q = (p + d) % N_DEV
            send_ref[d - 1, :, :] = chunk_partial(q).astype(jnp.bfloat16)

        barrier = pltpu.get_barrier_semaphore()
        for d in range(1, N_DEV):
            pl.semaphore_signal(
                barrier, inc=1,
                device_id=((p + d) % N_DEV,),
                device_id_type=pl.DeviceIdType.MESH,
            )
        pl.semaphore_wait(barrier, N_DEV - 1)

        rdmas = []
        for d in (2, 1, 3):
            q = (p + d) % N_DEV
            rdma = pltpu.make_async_remote_copy(
                src_ref=send_ref.at[d - 1],
                dst_ref=recv_ref.at[N_DEV - 1 - d],
                send_sem=send_sems.at[d - 1],
                recv_sem=recv_sems.at[N_DEV - 1 - d],
                device_id=(q,),
                device_id_type=pl.DeviceIdType.MESH,
            )
            rdma.start()
            rdmas.append(rdma)

        acc = chunk_partial(p)
        for rdma, d in zip(rdmas, (2, 1, 3)):
            rdma.wait()
            acc = acc + recv_ref[N_DEV - 1 - d, :, :].astype(jnp.float32)
        out_ref[:, :] = acc

    return pl.pallas_call(
        body,
        out_shape=jax.ShapeDtypeStruct((m_out, n), jnp.float32),
        in_specs=[
            pl.BlockSpec(memory_space=pltpu.VMEM),
            pl.BlockSpec(memory_space=pltpu.VMEM),
        ],
        out_specs=pl.BlockSpec(memory_space=pltpu.VMEM),
        scratch_shapes=[
            pltpu.VMEM((m, n), jnp.float32),
            pltpu.VMEM((N_DEV - 1, m_out, n), jnp.bfloat16),
            pltpu.VMEM((N_DEV - 1, m_out, n), jnp.bfloat16),
            pltpu.SemaphoreType.DMA((N_DEV - 1,)),
            pltpu.SemaphoreType.DMA((N_DEV - 1,)),
        ],
        compiler_params=pltpu.CompilerParams(collective_id=0),
    )(A, B)
